# baseline (speedup 1.0000x reference)
import functools

import jax
import jax.numpy as jnp
from jax import lax
from jax.experimental import pallas as pl
from jax.experimental.pallas import tpu as pltpu
from jax.experimental.pallas import tpu_sc as plsc

_B = 4096
_D = 64
_K = 5
_NC = 2
_NS = 16
_NW = _NC * _NS
_CHUNK = 128


def _sc_gather(in_embed, out_embed, idx_t, idx_cn):
    b = idx_t.shape[0]
    cn = idx_cn.shape[0]
    t_per_w = b // _NW
    cn_per_w = cn // _NW
    mesh = plsc.VectorSubcoreMesh(core_axis_name="c", subcore_axis_name="s")

    @functools.partial(
        pl.kernel,
        mesh=mesh,
        compiler_params=pltpu.CompilerParams(use_tc_tiling_on_sc=False),
        out_type=[
            jax.ShapeDtypeStruct((b, _D), jnp.float32),
            jax.ShapeDtypeStruct((cn, _D), jnp.float32),
        ],
        scratch_types=[
            pltpu.VMEM((t_per_w,), jnp.int32),
            pltpu.VMEM((cn_per_w,), jnp.int32),
            pltpu.VMEM((t_per_w, _D), jnp.float32),
            pltpu.VMEM((cn_per_w, _D), jnp.float32),
            pltpu.SemaphoreType.DMA,
        ],
    )
    def gather_kernel(in_hbm, out_hbm, it_hbm, icn_hbm, t_out, cn_out,
                      it_v, icn_v, t_rows, cn_rows, sem):
        wid = lax.axis_index("s") * _NC + lax.axis_index("c")
        tb = wid * t_per_w
        cb = wid * cn_per_w
        pltpu.sync_copy(it_hbm.at[pl.ds(tb, t_per_w)], it_v)
        pltpu.sync_copy(icn_hbm.at[pl.ds(cb, cn_per_w)], icn_v)
        copies = []
        for j in range(t_per_w // _CHUNK):
            copies.append(pltpu.async_copy(
                in_hbm.at[it_v.at[pl.ds(j * _CHUNK, _CHUNK)]],
                t_rows.at[pl.ds(j * _CHUNK, _CHUNK)], sem))
        for j in range(cn_per_w // _CHUNK):
            copies.append(pltpu.async_copy(
                out_hbm.at[icn_v.at[pl.ds(j * _CHUNK, _CHUNK)]],
                cn_rows.at[pl.ds(j * _CHUNK, _CHUNK)], sem))
        for c in copies:
            c.wait()
        pltpu.sync_copy(t_rows, t_out.at[pl.ds(tb, t_per_w)])
        pltpu.sync_copy(cn_rows, cn_out.at[pl.ds(cb, cn_per_w)])

    return gather_kernel(in_embed, out_embed, idx_t, idx_cn)


def _tc_loss(t_rows, cn_rows):
    def body(t_ref, cn_ref, o_ref):
        t = t_ref[...]
        c = cn_ref[0:_B, :]
        pos = jnp.sum(t * c, axis=1)
        acc = jnp.log(jax.nn.sigmoid(pos))
        for k in range(_K):
            n = cn_ref[_B * (k + 1):_B * (k + 2), :]
            s = jnp.sum(t * n, axis=1)
            acc = acc + jnp.log(jax.nn.sigmoid(-s))
        o_ref[0, 0] = -jnp.sum(acc) / _B

    out = pl.pallas_call(
        body,
        out_shape=jax.ShapeDtypeStruct((1, 1), jnp.float32),
        out_specs=pl.BlockSpec(memory_space=pltpu.SMEM),
    )(t_rows, cn_rows)
    return out[0, 0]


def kernel(target, context, neg_samples, in_embed, out_embed):
    idx_t = target.astype(jnp.int32)
    idx_cn = jnp.concatenate(
        [context.astype(jnp.int32), neg_samples.astype(jnp.int32).T.reshape(-1)])
    t_rows, cn_rows = _sc_gather(in_embed, out_embed, idx_t, idx_cn)
    return _tc_loss(t_rows, cn_rows)
